# baseline (speedup 1.0000x reference)
import functools

import jax
import jax.numpy as jnp
from jax import lax
from jax.experimental import pallas as pl

_TOPK = 0.25


def _topk_mask_kernel(x_ref, o_ref, *, k):
    x = x_ref[...]
    bits = lax.bitcast_convert_type(jnp.abs(x), jnp.int32)
    cols = (1, 1) + bits.shape[2:]

    hi16 = ((bits >> 15) - 32768).astype(jnp.int16)

    def body1(i, c):
        lo, hi = c
        mid = lo + ((hi - lo) >> 1)
        mid16 = (mid - 32768).astype(jnp.int16)
        cnt = jnp.sum((hi16 >= mid16).astype(jnp.int16), axis=1, keepdims=True)
        ge = cnt >= jnp.int16(k)
        return jnp.where(ge, mid, lo), jnp.where(ge, hi, mid)

    lo1, _ = lax.fori_loop(
        0, 16, body1,
        (jnp.zeros(cols, jnp.int32), jnp.full(cols, 65536, jnp.int32)),
    )
    t16 = (lo1 - 32768).astype(jnp.int16)

    low15 = (bits & 0x7FFF).astype(jnp.int16)
    z = jnp.where(
        hi16 > t16,
        jnp.int16(32767),
        jnp.where(hi16 == t16, low15, jnp.int16(-1)),
    )

    def body2(i, c):
        lo, hi = c
        mid = lo + ((hi - lo) >> 1)
        cnt = jnp.sum((z >= mid.astype(jnp.int16)).astype(jnp.int16),
                      axis=1, keepdims=True)
        ge = cnt >= jnp.int16(k)
        return jnp.where(ge, mid, lo), jnp.where(ge, hi, mid)

    lo2, _ = lax.fori_loop(
        0, 15, body2,
        (jnp.zeros(cols, jnp.int32), jnp.full(cols, 32768, jnp.int32)),
    )

    thr = (lo1 << 15) | lo2
    o_ref[...] = jnp.where(bits >= thr, x, jnp.zeros_like(x))


def kernel(x, tau):
    n, c, h, w = x.shape
    k = max(int(_TOPK * c), 1)
    p = h * w
    xr = x.reshape(n, c, p)
    sparse = pl.pallas_call(
        functools.partial(_topk_mask_kernel, k=k),
        out_shape=jax.ShapeDtypeStruct((n, c, p), x.dtype),
        grid=(n,),
        in_specs=[pl.BlockSpec((1, c, p), lambda i: (i, 0, 0))],
        out_specs=pl.BlockSpec((1, c, p), lambda i: (i, 0, 0)),
    )(xr).reshape(n, c, h, w)
    tau_arr = jnp.asarray(tau)
    tau_f = tau_arr.astype(x.dtype)
    blended = sparse * tau_f + x * (1.0 - tau_f)
    return jnp.where(tau_arr == 1, sparse, blended)


# speedup vs baseline: 1.9966x; 1.9966x over previous
import functools

import jax
import jax.numpy as jnp
from jax import lax
from jax.experimental import pallas as pl
from jax.experimental.pallas import tpu as pltpu

_TOPK = 0.25


def _topk_mask_kernel(x_ref, o_ref, bits_ref, *, k):
    x = x_ref[0]
    bits_ref[...] = lax.bitcast_convert_type(jnp.abs(x), jnp.int32)
    p = x.shape[1]
    lo0 = jnp.zeros((1, p), jnp.int32)
    hi0 = jnp.full((1, p), jnp.int32(0x7FFFFFFF), jnp.int32)

    def body(i, c):
        lo, hi = c
        mid = lo + ((hi - lo) >> 1)
        cnt = jnp.sum((bits_ref[...] >= mid).astype(jnp.int32), axis=0,
                      keepdims=True)
        ge = cnt >= k
        return jnp.where(ge, mid, lo), jnp.where(ge, hi, mid)

    lo, _ = lax.fori_loop(0, 31, body, (lo0, hi0))
    o_ref[0] = jnp.where(bits_ref[...] >= lo, x, jnp.zeros_like(x))


def kernel(x, tau):
    n, c, h, w = x.shape
    k = max(int(_TOPK * c), 1)
    p = h * w
    xr = x.reshape(n, c, p)
    sparse = pl.pallas_call(
        functools.partial(_topk_mask_kernel, k=k),
        out_shape=jax.ShapeDtypeStruct((n, c, p), x.dtype),
        grid=(n,),
        in_specs=[pl.BlockSpec((1, c, p), lambda i: (i, 0, 0))],
        out_specs=pl.BlockSpec((1, c, p), lambda i: (i, 0, 0)),
        scratch_shapes=[pltpu.VMEM((c, p), jnp.int32)],
    )(xr).reshape(n, c, h, w)
    tau_arr = jnp.asarray(tau)
    tau_f = tau_arr.astype(x.dtype)
    blended = sparse * tau_f + x * (1.0 - tau_f)
    return jnp.where(tau_arr == 1, sparse, blended)
